# baseline (device time: 25141 ns/iter reference)
import jax
import jax.numpy as jnp
from jax import lax
from jax.experimental import pallas as pl
from jax.experimental.pallas import tpu as pltpu

N_DEV = 4
B, SQ, SKV = 2, 256, 256
HQ_LOC, DH = 4, 64
DM = 512
BLK = 64
RC = SQ // 2


def kernel(x, Wq, K_ext, V_ext, Wo):
    def body(x_ref, wq_ref, k_hbm, v_hbm, wo_ref, out_ref,
             k_ref, v_ref, send0_ref, send1_ref, recv_ref,
             load_sems, send_sems, recv_sems):
        my_p = lax.axis_index("i")
        partner1 = my_p ^ 1
        partner2 = 3 - my_p

        loads = {0: [], 1: []}
        for b in range(B):
            for h in range(HQ_LOC):
                li = b * HQ_LOC + h
                for src, dst, off in ((k_hbm, k_ref, 0), (v_hbm, v_ref, 8)):
                    cp = pltpu.make_async_copy(
                        src.at[b, :, my_p * HQ_LOC + h, :],
                        dst.at[b, h],
                        load_sems.at[off + li],
                    )
                    cp.start()
                    loads[b].append(cp)
        loads_pending = {0: True, 1: True}

        barrier = pltpu.get_barrier_semaphore()
        for nbr in (partner1, partner2):
            pl.semaphore_signal(
                barrier, inc=1,
                device_id=(nbr,), device_id_type=pl.DeviceIdType.MESH,
            )

        order = [(0, 0), (1, 0), (0, 1), (1, 1)]

        def compute_chunk(b, r):
            ci = b * 2 + r
            kn = (r + 1) * RC
            rows = slice(r * RC, (r + 1) * RC)
            qb = lax.broadcasted_iota(jnp.int32, (RC, kn), 0) // BLK + (
                r * RC // BLK)
            kb = lax.broadcasted_iota(jnp.int32, (RC, kn), 1) // BLK
            mask = kb <= qb
            q_all = jnp.dot(x_ref[b, rows], wq_ref[...],
                            preferred_element_type=jnp.float32) * 0.125
            if loads_pending[b]:
                for cp in loads[b]:
                    cp.wait()
                loads_pending[b] = False
            acc = jnp.zeros((RC, DM), jnp.float32)
            for h in range(HQ_LOC):
                q = q_all[:, h * DH:(h + 1) * DH]
                k = k_ref[b, h, :kn]
                v = v_ref[b, h, :kn]
                s = lax.dot_general(
                    q, k, (((1,), (1,)), ((), ())),
                    preferred_element_type=jnp.float32)
                w = jnp.exp(jnp.where(mask, s, -1e9))
                w = w / jnp.sum(w, axis=-1, keepdims=True)
                ctx = jnp.dot(w, v, preferred_element_type=jnp.float32)
                acc = acc + jnp.dot(ctx, wo_ref[h * DH:(h + 1) * DH, :],
                                    preferred_element_type=jnp.float32)
            out_ref[b, rows] = acc
            send0_ref[ci] = acc.astype(jnp.bfloat16)

        def start_ex(stage, ci, partner, src_ref):
            idx = stage * 4 + ci
            rdma = pltpu.make_async_remote_copy(
                src_ref=src_ref.at[ci],
                dst_ref=recv_ref.at[idx],
                send_sem=send_sems.at[idx],
                recv_sem=recv_sems.at[idx],
                device_id=(partner,),
                device_id_type=pl.DeviceIdType.MESH,
            )
            rdma.start()
            return rdma

        s0 = {}
        s1 = {}

        def finish0(b, r):
            ci = b * 2 + r
            s0[ci].wait()
            rows = slice(r * RC, (r + 1) * RC)
            tmp = out_ref[b, rows] + recv_ref[ci].astype(jnp.float32)
            out_ref[b, rows] = tmp
            send1_ref[ci] = tmp.astype(jnp.bfloat16)
            s1[ci] = start_ex(1, ci, partner2 if b == 0 else partner1,
                              send1_ref)

        for i, (b, r) in enumerate(order):
            compute_chunk(b, r)
            if i == 0:
                pl.semaphore_wait(barrier, 2)
            s0[b * 2 + r] = start_ex(0, b * 2 + r,
                                     partner1 if b == 0 else partner2,
                                     send0_ref)
            if i >= 2:
                finish0(*order[i - 2])
        for (b, r) in order[-2:]:
            finish0(b, r)

        for (b, r) in order:
            ci = b * 2 + r
            s1[ci].wait()
            rows = slice(r * RC, (r + 1) * RC)
            out_ref[b, rows] = (
                out_ref[b, rows] + recv_ref[4 + ci].astype(jnp.float32))

    return pl.pallas_call(
        body,
        out_shape=jax.ShapeDtypeStruct((B, SQ, DM), jnp.float32),
        in_specs=[
            pl.BlockSpec(memory_space=pltpu.VMEM),
            pl.BlockSpec(memory_space=pltpu.VMEM),
            pl.BlockSpec(memory_space=pl.ANY),
            pl.BlockSpec(memory_space=pl.ANY),
            pl.BlockSpec(memory_space=pltpu.VMEM),
        ],
        out_specs=pl.BlockSpec(memory_space=pltpu.VMEM),
        scratch_shapes=[
            pltpu.VMEM((B, HQ_LOC, SKV, DH), jnp.float32),
            pltpu.VMEM((B, HQ_LOC, SKV, DH), jnp.float32),
            pltpu.VMEM((4, RC, DM), jnp.bfloat16),
            pltpu.VMEM((4, RC, DM), jnp.bfloat16),
            pltpu.VMEM((8, RC, DM), jnp.bfloat16),
            pltpu.SemaphoreType.DMA((16,)),
            pltpu.SemaphoreType.DMA((8,)),
            pltpu.SemaphoreType.DMA((8,)),
        ],
        compiler_params=pltpu.CompilerParams(collective_id=0),
    )(x, Wq, K_ext, V_ext, Wo)


# device time: 19262 ns/iter; 1.3052x vs baseline; 1.3052x over previous
import jax
import jax.numpy as jnp
from jax import lax
from jax.experimental import pallas as pl
from jax.experimental.pallas import tpu as pltpu

N_DEV = 4
B, SQ, SKV = 2, 256, 256
HQ_LOC, DH = 4, 64
DM = 512
BLK = 64
RC = SQ // 2


def kernel(x, Wq, K_ext, V_ext, Wo):
    p = lax.axis_index("i")
    K_loc = lax.dynamic_slice_in_dim(K_ext, p * HQ_LOC, HQ_LOC, axis=2)
    V_loc = lax.dynamic_slice_in_dim(V_ext, p * HQ_LOC, HQ_LOC, axis=2)
    K_loc = jnp.transpose(K_loc, (0, 2, 1, 3))
    V_loc = jnp.transpose(V_loc, (0, 2, 1, 3))

    def body(x_ref, wq_ref, k_ref, v_ref, wo_ref, out_ref,
             send0_ref, send1_ref, recv_ref, send_sems, recv_sems):
        my_p = lax.axis_index("i")
        partner1 = my_p ^ 1
        partner2 = 3 - my_p

        barrier = pltpu.get_barrier_semaphore()
        for nbr in (partner1, partner2):
            pl.semaphore_signal(
                barrier, inc=1,
                device_id=(nbr,), device_id_type=pl.DeviceIdType.MESH,
            )

        order = [(0, 0), (1, 0), (0, 1), (1, 1)]

        def compute_chunk(b, r):
            ci = b * 2 + r
            kn = (r + 1) * RC
            rows = slice(r * RC, (r + 1) * RC)
            qb = lax.broadcasted_iota(jnp.int32, (RC, kn), 0) // BLK + (
                r * RC // BLK)
            kb = lax.broadcasted_iota(jnp.int32, (RC, kn), 1) // BLK
            mask = kb <= qb
            q_all = jnp.dot(x_ref[b, rows], wq_ref[...],
                            preferred_element_type=jnp.float32) * 0.125
            acc = jnp.zeros((RC, DM), jnp.float32)
            for h in range(HQ_LOC):
                q = q_all[:, h * DH:(h + 1) * DH]
                k = k_ref[b, h, :kn]
                v = v_ref[b, h, :kn]
                s = lax.dot_general(
                    q, k, (((1,), (1,)), ((), ())),
                    preferred_element_type=jnp.float32)
                w = jnp.exp(jnp.where(mask, s, -1e9))
                w = w / jnp.sum(w, axis=-1, keepdims=True)
                ctx = jnp.dot(w, v, preferred_element_type=jnp.float32)
                acc = acc + jnp.dot(ctx, wo_ref[h * DH:(h + 1) * DH, :],
                                    preferred_element_type=jnp.float32)
            out_ref[b, rows] = acc
            send0_ref[ci] = acc.astype(jnp.bfloat16)

        def start_ex(stage, ci, partner, src_ref):
            idx = stage * 4 + ci
            rdma = pltpu.make_async_remote_copy(
                src_ref=src_ref.at[ci],
                dst_ref=recv_ref.at[idx],
                send_sem=send_sems.at[idx],
                recv_sem=recv_sems.at[idx],
                device_id=(partner,),
                device_id_type=pl.DeviceIdType.MESH,
            )
            rdma.start()
            return rdma

        s0 = {}
        s1 = {}

        def finish0(b, r):
            ci = b * 2 + r
            s0[ci].wait()
            rows = slice(r * RC, (r + 1) * RC)
            tmp = out_ref[b, rows] + recv_ref[ci].astype(jnp.float32)
            out_ref[b, rows] = tmp
            send1_ref[ci] = tmp.astype(jnp.bfloat16)
            s1[ci] = start_ex(1, ci, partner2 if b == 0 else partner1,
                              send1_ref)

        for i, (b, r) in enumerate(order):
            compute_chunk(b, r)
            if i == 0:
                pl.semaphore_wait(barrier, 2)
            s0[b * 2 + r] = start_ex(0, b * 2 + r,
                                     partner1 if b == 0 else partner2,
                                     send0_ref)
            if i >= 2:
                finish0(*order[i - 2])
        for (b, r) in order[-2:]:
            finish0(b, r)

        for (b, r) in order:
            ci = b * 2 + r
            s1[ci].wait()
            rows = slice(r * RC, (r + 1) * RC)
            out_ref[b, rows] = (
                out_ref[b, rows] + recv_ref[4 + ci].astype(jnp.float32))

    return pl.pallas_call(
        body,
        out_shape=jax.ShapeDtypeStruct((B, SQ, DM), jnp.float32),
        in_specs=[pl.BlockSpec(memory_space=pltpu.VMEM)] * 5,
        out_specs=pl.BlockSpec(memory_space=pltpu.VMEM),
        scratch_shapes=[
            pltpu.VMEM((4, RC, DM), jnp.bfloat16),
            pltpu.VMEM((4, RC, DM), jnp.bfloat16),
            pltpu.VMEM((8, RC, DM), jnp.bfloat16),
            pltpu.SemaphoreType.DMA((8,)),
            pltpu.SemaphoreType.DMA((8,)),
        ],
        compiler_params=pltpu.CompilerParams(collective_id=0),
    )(x, Wq, K_loc, V_loc, Wo)
